# Initial kernel scaffold; baseline (speedup 1.0000x reference)
import jax
import jax.numpy as jnp
from jax.experimental import pallas as pl


def kernel(x, edge_index, W0, b0, W1, b1):
    raise NotImplementedError("write your pallas kernel here")


# speedup vs baseline: 9.6519x; 9.6519x over previous
import functools

import jax
import jax.numpy as jnp
from jax import lax
from jax.experimental import pallas as pl
from jax.experimental.pallas import tpu as pltpu
from jax.experimental.pallas import tpu_sc as plsc

NC = 2
NS = 16
TILES = NC * NS
C = 128
HW = 64


def _sc_mesh():
    return plsc.VectorSubcoreMesh(core_axis_name="c", subcore_axis_name="s")


def _row_chunks(rpw):
    sizes = []
    left = rpw
    while left > 0:
        sz = min(C, left)
        sizes.append(sz)
        left -= sz
    return sizes


def _make_deg_kernel(NR, NCHD):
    rpw = NR // NS
    chunks = _row_chunks(rpw)

    @functools.partial(
        pl.kernel,
        out_type=jax.ShapeDtypeStruct((NC, NR, 16), jnp.float32),
        mesh=_sc_mesh(),
        scratch_types=[
            pltpu.VMEM((NCHD, C), jnp.int32),
            pltpu.VMEM((C, 16), jnp.float32),
            pltpu.VMEM((C, 16), jnp.float32),
            pltpu.VMEM_SHARED((NR, 16), jnp.float32),
        ],
        compiler_params=pltpu.CompilerParams(use_tc_tiling_on_sc=False),
    )
    def deg_kernel(src_hbm, ones_hbm, zeros_hbm, out_hbm, src_v, ones_v,
                   stage_v, deg_sh):
        c = lax.axis_index("c")
        s = lax.axis_index("s")
        wid = s * NC + c
        pltpu.sync_copy(ones_hbm, ones_v)
        pltpu.sync_copy(zeros_hbm, stage_v)
        off = 0
        for sz in chunks:
            pltpu.sync_copy(stage_v.at[pl.ds(0, sz)],
                            deg_sh.at[pl.ds(s * rpw + off, sz)])
            off += sz
        plsc.subcore_barrier()
        pltpu.sync_copy(src_hbm.at[wid], src_v)

        def body(j, carry):
            pltpu.sync_copy(ones_v, deg_sh.at[src_v.at[j]], add=True)
            return carry

        lax.fori_loop(0, NCHD, body, 0)
        plsc.subcore_barrier()
        off = 0
        for sz in chunks:
            o = s * rpw + off
            pltpu.sync_copy(deg_sh.at[pl.ds(o, sz)], stage_v.at[pl.ds(0, sz)])
            pltpu.sync_copy(stage_v.at[pl.ds(0, sz)], out_hbm.at[c, pl.ds(o, sz)])
            off += sz

    return deg_kernel


def _make_prop_kernel(NR, NCH):
    rpw = NR // NS
    chunks = _row_chunks(rpw)

    @functools.partial(
        pl.kernel,
        out_type=jax.ShapeDtypeStruct((NR, 128), jnp.float32),
        mesh=_sc_mesh(),
        scratch_types=[
            pltpu.VMEM((NCH, C), jnp.int32),
            pltpu.VMEM((NCH, C), jnp.int32),
            pltpu.VMEM((2, C, HW), jnp.float32),
            pltpu.VMEM((C, HW), jnp.float32),
            pltpu.VMEM_SHARED((NR, HW), jnp.float32),
            pltpu.SemaphoreType.DMA,
            pltpu.SemaphoreType.DMA,
        ],
        compiler_params=pltpu.CompilerParams(use_tc_tiling_on_sc=False),
    )
    def prop_kernel(g2_hbm, srcx_hbm, dst_hbm, zeros_hbm, out_hbm, src_v,
                    dst_v, rows_v, stage_v, acc_sh, gsem, hsem):
        c = lax.axis_index("c")
        s = lax.axis_index("s")
        pltpu.sync_copy(zeros_hbm, stage_v)
        off = 0
        for sz in chunks:
            pltpu.sync_copy(stage_v.at[pl.ds(0, sz)],
                            acc_sh.at[pl.ds(s * rpw + off, sz)])
            off += sz
        plsc.subcore_barrier()
        pltpu.sync_copy(srcx_hbm.at[c, s], src_v)
        pltpu.sync_copy(dst_hbm.at[s], dst_v)

        buf_a = rows_v.at[0]
        buf_b = rows_v.at[1]
        pltpu.async_copy(g2_hbm.at[src_v.at[0]], buf_a, gsem)

        def body(i, carry):
            j = 2 * i
            pltpu.async_copy(g2_hbm.at[src_v.at[j + 1]], buf_b, hsem)
            pltpu.make_async_copy(g2_hbm.at[src_v.at[j]], buf_a, gsem).wait()
            pltpu.sync_copy(buf_a, acc_sh.at[dst_v.at[j]], add=True)

            @pl.when(j + 2 < NCH)
            def _():
                pltpu.async_copy(g2_hbm.at[src_v.at[j + 2]], buf_a, gsem)

            pltpu.make_async_copy(g2_hbm.at[src_v.at[j + 1]], buf_b, hsem).wait()
            pltpu.sync_copy(buf_b, acc_sh.at[dst_v.at[j + 1]], add=True)
            return carry

        lax.fori_loop(0, NCH // 2, body, 0)
        plsc.subcore_barrier()
        off = 0
        for sz in chunks:
            o = s * rpw + off
            pltpu.sync_copy(acc_sh.at[pl.ds(o, sz)], stage_v.at[pl.ds(0, sz)])
            pltpu.sync_copy(stage_v.at[pl.ds(0, sz)],
                            out_hbm.at[pl.ds(o, sz), pl.ds(c * HW, HW)])
            off += sz

    return prop_kernel


_R = 2048


def _blk(imap=None):
    return pl.BlockSpec((_R, 128), imap or (lambda i: (i, 0)))


def _full(shape):
    return pl.BlockSpec(shape, lambda i: tuple(0 for _ in shape))


def _tca(dis128, x, w00, NR):
    def body(dis_ref, x_ref, w_ref, g0_ref, acc_ref):
        xv = x_ref[...]
        g0_ref[...] = dis_ref[...] * xv
        acc_ref[...] = jnp.dot(xv, w_ref[...], preferred_element_type=jnp.float32)

    return pl.pallas_call(
        body,
        grid=(pl.cdiv(NR, _R),),
        in_specs=[_blk(), _blk(), _full((128, 128))],
        out_specs=[_blk(), _blk()],
        out_shape=[
            jax.ShapeDtypeStruct((NR, 128), jnp.float32),
            jax.ShapeDtypeStruct((NR, 128), jnp.float32),
        ],
    )(dis128, x, w00)


def _tcb(up, dis128, acc, w, NR):
    def body(up_ref, dis_ref, acc_ref, w_ref, aout_ref, g_ref):
        dis = dis_ref[...]
        tx = -dis * up_ref[...]
        aout_ref[...] = acc_ref[...] + jnp.dot(
            tx, w_ref[...], preferred_element_type=jnp.float32)
        g_ref[...] = dis * tx

    return pl.pallas_call(
        body,
        grid=(pl.cdiv(NR, _R),),
        in_specs=[_blk(), _blk(), _blk(), _full((128, 128))],
        out_specs=[_blk(), _blk()],
        out_shape=[
            jax.ShapeDtypeStruct((NR, 128), jnp.float32),
            jax.ShapeDtypeStruct((NR, 128), jnp.float32),
        ],
    )(up, dis128, acc, w)


def _tcc(up, dis128, x, acc, w02, b0, w10, NR):
    def body(up_ref, dis_ref, x_ref, acc_ref, w02_ref, b0_ref,
             w10_ref, h_ref, g_ref, acc2_ref):
        dis = dis_ref[...]
        tx2 = 2.0 * (-dis * up_ref[...]) - x_ref[...]
        out1 = acc_ref[...] + jnp.dot(
            tx2, w02_ref[...], preferred_element_type=jnp.float32) + b0_ref[...]
        h = jnp.maximum(out1, 0.0)
        h_ref[...] = h
        g_ref[...] = dis * h
        acc2_ref[...] = jnp.dot(h, w10_ref[...], preferred_element_type=jnp.float32)

    return pl.pallas_call(
        body,
        grid=(pl.cdiv(NR, _R),),
        in_specs=[_blk(), _blk(), _blk(), _blk(),
                  _full((128, 128)), _full((1, 128)), _full((128, 128))],
        out_specs=[_blk(), _blk(), _blk()],
        out_shape=[
            jax.ShapeDtypeStruct((NR, 128), jnp.float32),
            jax.ShapeDtypeStruct((NR, 128), jnp.float32),
            jax.ShapeDtypeStruct((NR, 128), jnp.float32),
        ],
    )(up, dis128, x, acc, w02, b0, w10)


def _tce(up, dis128, h, acc, w12, b1, NR):
    def body(up_ref, dis_ref, h_ref, acc_ref, w_ref, b_ref, out_ref):
        tx2 = 2.0 * (-dis_ref[...] * up_ref[...]) - h_ref[...]
        out_ref[...] = acc_ref[...] + jnp.dot(
            tx2, w_ref[...], preferred_element_type=jnp.float32) + b_ref[...]

    return pl.pallas_call(
        body,
        grid=(pl.cdiv(NR, _R),),
        in_specs=[_blk(), _blk(), _blk(), _blk(),
                  _full((128, 128)), _full((1, 128))],
        out_specs=_blk(),
        out_shape=jax.ShapeDtypeStruct((NR, 128), jnp.float32),
    )(up, dis128, h, acc, w12, b1)


def kernel(x, edge_index, W0, b0, W1, b1):
    N, D = x.shape
    E = edge_index.shape[1]

    NR = ((N + 1 + 8 * NS - 1) // (8 * NS)) * (8 * NS)
    src = edge_index[0]
    dst = edge_index[1]

    ECH = NS * C
    EPAD = ((E + ECH - 1) // ECH) * ECH
    NCH = EPAD // ECH
    if NCH % 2:
        NCH += 1
        EPAD = NCH * ECH
    srcp = jnp.pad(src, (0, EPAD - E), constant_values=N).reshape(NS, NCH, C)
    srcx = jnp.stack([srcp * 2, srcp * 2 + 1])
    dst3 = jnp.pad(dst, (0, EPAD - E), constant_values=N).reshape(NS, NCH, C)

    ECHD = TILES * C
    EPADD = ((E + ECHD - 1) // ECHD) * ECHD
    NCHD = EPADD // ECHD
    srcd = jnp.pad(src, (0, EPADD - E), constant_values=N).reshape(
        TILES, NCHD, C)

    x_p = jnp.pad(x, ((0, NR - N), (0, 0)))
    ones16 = jnp.ones((C, 16), jnp.float32)
    zeros16 = jnp.zeros((C, 16), jnp.float32)
    zeros64 = jnp.zeros((C, HW), jnp.float32)
    b0r = b0.reshape(1, 128)
    b1r = b1.reshape(1, 128)

    deg_kernel = _make_deg_kernel(NR, NCHD)
    prop_kernel = _make_prop_kernel(NR, NCH)

    def prop(g):
        return prop_kernel(g.reshape(2 * NR, HW), srcx, dst3, zeros64)

    degp = deg_kernel(srcd, ones16, zeros16)
    deg = degp[0, :, 0] + degp[1, :, 0]
    dis = jnp.where(deg > 0, 1.0 / jnp.sqrt(jnp.where(deg > 0, deg, 1.0)), 0.0)
    dis128 = jnp.broadcast_to(dis[:, None], (NR, 128))

    g0, acc0 = _tca(dis128, x_p, W0[0], NR)
    acc1, g1 = _tcb(prop(g0), dis128, acc0, W0[1], NR)
    h, g2, acc2 = _tcc(prop(g1), dis128, x_p, acc1, W0[2], b0r, W1[0], NR)
    acc3, g3 = _tcb(prop(g2), dis128, acc2, W1[1], NR)
    out = _tce(prop(g3), dis128, h, acc3, W1[2], b1r, NR)

    return out[:N]
